# Initial kernel scaffold; baseline (speedup 1.0000x reference)
import jax
import jax.numpy as jnp
from jax.experimental import pallas as pl


def kernel(x, edge_index, W1, b1, W2, b2):
    raise NotImplementedError("write your pallas kernel here")


# speedup vs baseline: 9.6681x; 9.6681x over previous
import functools

import jax
import jax.numpy as jnp
from jax import lax
from jax.experimental import pallas as pl
from jax.experimental.pallas import tpu as pltpu
from jax.experimental.pallas import tpu_sc as plsc

N_NODES = 10000
D_IN = 128
D_HID = 128
D_OUT = 64
DW = 64

NC = 2
NS = 16
LANES = 16
NW = NC * NS
CHUNK = 128
CHUNKS_PER_TILE = 80
E_TILE = CHUNK * CHUNKS_PER_TILE
E_PAD = NW * E_TILE
N_PAD = 10240
SLAB = N_PAD // NS


@functools.lru_cache(maxsize=None)
def _get_mesh():
    return plsc.VectorSubcoreMesh(core_axis_name="c", subcore_axis_name="s",
                                  num_cores=NC, num_subcores=NS)


def _deg_body(dst_hbm, out_hbm, dst_v, zbuf, obuf, deg_sh):
    c = lax.axis_index("c")
    s = lax.axis_index("s")
    wid = s * NC + c
    zeros = jnp.zeros((LANES,), jnp.float32)
    ones = jnp.full((LANES,), 1.0, jnp.float32)
    for j in range(CHUNK // LANES):
        zbuf[pl.ds(j * LANES, LANES)] = zeros
        obuf[pl.ds(j * LANES, LANES)] = ones

    def zslab(i, carry):
        pltpu.sync_copy(zbuf, deg_sh.at[pl.ds(s * SLAB + i * CHUNK, CHUNK)])
        return carry

    lax.fori_loop(0, SLAB // CHUNK, zslab, 0)
    pltpu.sync_copy(dst_hbm.at[wid], dst_v)
    plsc.subcore_barrier()

    def body(k, carry):
        pltpu.sync_copy(obuf, deg_sh.at[dst_v.at[k]], add=True)
        return carry

    lax.fori_loop(0, CHUNKS_PER_TILE, body, 0)
    plsc.subcore_barrier()
    pltpu.sync_copy(deg_sh.at[pl.ds(s * SLAB, SLAB)],
                    out_hbm.at[c, pl.ds(s * SLAB, SLAB)])


@functools.lru_cache(maxsize=None)
def _deg_kernel():
    return pl.kernel(
        _deg_body,
        out_type=jax.ShapeDtypeStruct((NC, N_PAD), jnp.float32),
        mesh=_get_mesh(),
        scratch_types=[
            pltpu.VMEM((CHUNKS_PER_TILE, CHUNK), jnp.int32),
            pltpu.VMEM((CHUNK,), jnp.float32),
            pltpu.VMEM((CHUNK,), jnp.float32),
            pltpu.VMEM_SHARED((N_PAD,), jnp.float32),
        ],
        compiler_params=pltpu.CompilerParams(use_tc_tiling_on_sc=False),
    )


def _zero_gbuf0(gbuf):
    zeros = jnp.zeros((LANES,), jnp.float32)

    def zrow(i, carry):
        for j in range(DW // LANES):
            gbuf[0, i, pl.ds(j * LANES, LANES)] = zeros
        return carry

    lax.fori_loop(0, CHUNK, zrow, 0)


def _agg_pass(c, s, src_v, dst_v, gbuf, acc_sh, sem, h_hbm, out_hbm):
    def zslab(i, carry):
        pltpu.sync_copy(gbuf.at[0],
                        acc_sh.at[pl.ds(s * SLAB + i * CHUNK, CHUNK)])
        return carry

    lax.fori_loop(0, SLAB // CHUNK, zslab, 0)
    plsc.subcore_barrier()

    def body(k, carry):
        pltpu.async_copy(h_hbm.at[src_v.at[k]], gbuf.at[1], sem).wait()
        pltpu.sync_copy(gbuf.at[1], acc_sh.at[dst_v.at[k]], add=True)
        return carry

    lax.fori_loop(0, CHUNKS_PER_TILE, body, 0)
    plsc.subcore_barrier()

    def wout(i, carry):
        pltpu.sync_copy(
            acc_sh.at[pl.ds(s * SLAB + i * CHUNK, CHUNK)],
            out_hbm.at[c, pl.ds(s * SLAB + i * CHUNK, CHUNK), :])
        return carry

    lax.fori_loop(0, SLAB // CHUNK, wout, 0)


def _agg2_body(src_hbm, dst_hbm, hl_hbm, hr_hbm, out_hbm,
               src_v, dst_v, gbuf, acc_sh, sem):
    c = lax.axis_index("c")
    s = lax.axis_index("s")
    wid = s * NC + c
    _zero_gbuf0(gbuf)
    pltpu.sync_copy(src_hbm.at[wid], src_v)
    pltpu.sync_copy(dst_hbm.at[wid], dst_v)
    _agg_pass(c, s, src_v, dst_v, gbuf, acc_sh, sem, hl_hbm, out_hbm.at[0])
    plsc.subcore_barrier()
    _agg_pass(c, s, src_v, dst_v, gbuf, acc_sh, sem, hr_hbm, out_hbm.at[1])


def _agg1_body(src_hbm, dst_hbm, h_hbm, out_hbm,
               src_v, dst_v, gbuf, acc_sh, sem):
    c = lax.axis_index("c")
    s = lax.axis_index("s")
    wid = s * NC + c
    _zero_gbuf0(gbuf)
    pltpu.sync_copy(src_hbm.at[wid], src_v)
    pltpu.sync_copy(dst_hbm.at[wid], dst_v)
    _agg_pass(c, s, src_v, dst_v, gbuf, acc_sh, sem, h_hbm, out_hbm)


_AGG_SCRATCH = (
    pltpu.VMEM((CHUNKS_PER_TILE, CHUNK), jnp.int32),
    pltpu.VMEM((CHUNKS_PER_TILE, CHUNK), jnp.int32),
    pltpu.VMEM((2, CHUNK, DW), jnp.float32),
    pltpu.VMEM_SHARED((N_PAD, DW), jnp.float32),
    pltpu.SemaphoreType.DMA,
)


@functools.lru_cache(maxsize=None)
def _agg2_kernel():
    return pl.kernel(
        _agg2_body,
        out_type=jax.ShapeDtypeStruct((2, NC, N_PAD, DW), jnp.float32),
        mesh=_get_mesh(),
        scratch_types=list(_AGG_SCRATCH),
        compiler_params=pltpu.CompilerParams(use_tc_tiling_on_sc=False),
    )


@functools.lru_cache(maxsize=None)
def _agg1_kernel():
    return pl.kernel(
        _agg1_body,
        out_type=jax.ShapeDtypeStruct((NC, N_PAD, DW), jnp.float32),
        mesh=_get_mesh(),
        scratch_types=list(_AGG_SCRATCH),
        compiler_params=pltpu.CompilerParams(use_tc_tiling_on_sc=False),
    )


BLK = 1024
GRID = N_PAD // BLK


def _tc_a_body(dp0_ref, dp1_ref, x_ref, w1_ref, hl_ref, hr_ref, dinv_ref):
    dinv = lax.rsqrt(dp0_ref[...] + dp1_ref[...] + 1.0)
    h = jnp.dot(x_ref[...], w1_ref[...],
                preferred_element_type=jnp.float32) * dinv
    hl_ref[...] = h[:, :DW]
    hr_ref[...] = h[:, DW:]
    dinv_ref[...] = dinv


def _tc_b_body(dinv_ref, hl_ref, hr_ref, pl0_ref, pl1_ref, pr0_ref, pr1_ref,
               b1_ref, w2_ref, h2s_ref):
    dinv = dinv_ref[...]
    zl = (pl0_ref[...] + pl1_ref[...] + hl_ref[...]) * dinv
    zr = (pr0_ref[...] + pr1_ref[...] + hr_ref[...]) * dinv
    z = jnp.maximum(jnp.concatenate([zl, zr], axis=1) + b1_ref[...], 0.0)
    h2s_ref[...] = jnp.dot(z, w2_ref[...],
                           preferred_element_type=jnp.float32) * dinv


def _tc_c_body(dinv_ref, h2s_ref, q0_ref, q1_ref, b2_ref, y_ref):
    t = (q0_ref[...] + q1_ref[...] + h2s_ref[...]) * dinv_ref[...] \
        + b2_ref[...]
    y_ref[...] = jax.nn.sigmoid(t)


def _row_spec(d):
    return pl.BlockSpec((BLK, d), lambda i: (i, 0))


def _full_spec(r, cdim):
    return pl.BlockSpec((r, cdim), lambda i: (0, 0))


_tc_a = pl.pallas_call(
    _tc_a_body,
    grid=(GRID,),
    in_specs=[_row_spec(1), _row_spec(1), _row_spec(D_IN),
              _full_spec(D_IN, D_HID)],
    out_specs=[_row_spec(DW), _row_spec(DW), _row_spec(1)],
    out_shape=[jax.ShapeDtypeStruct((N_PAD, DW), jnp.float32),
               jax.ShapeDtypeStruct((N_PAD, DW), jnp.float32),
               jax.ShapeDtypeStruct((N_PAD, 1), jnp.float32)],
)

_tc_b = pl.pallas_call(
    _tc_b_body,
    grid=(GRID,),
    in_specs=[_row_spec(1)] + [_row_spec(DW)] * 6 +
             [_full_spec(1, D_HID), _full_spec(D_HID, D_OUT)],
    out_specs=_row_spec(D_OUT),
    out_shape=jax.ShapeDtypeStruct((N_PAD, D_OUT), jnp.float32),
)

_tc_c = pl.pallas_call(
    _tc_c_body,
    grid=(GRID,),
    in_specs=[_row_spec(1), _row_spec(D_OUT), _row_spec(D_OUT),
              _row_spec(D_OUT), _full_spec(1, D_OUT)],
    out_specs=_row_spec(D_OUT),
    out_shape=jax.ShapeDtypeStruct((N_PAD, D_OUT), jnp.float32),
)


@jax.jit
def _run(x, edge_index, W1, b1, W2, b2):
    n_edges = edge_index.shape[1]
    src = edge_index[0].astype(jnp.int32)
    dst = edge_index[1].astype(jnp.int32)
    pad = E_PAD - n_edges
    src = jnp.concatenate([src, jnp.zeros((pad,), jnp.int32)])
    dst = jnp.concatenate([dst, jnp.full((pad,), N_NODES, jnp.int32)])
    src_r = src.reshape(NW, CHUNKS_PER_TILE, CHUNK)
    dst_r = dst.reshape(NW, CHUNKS_PER_TILE, CHUNK)
    x_pad = jnp.pad(x, ((0, N_PAD - x.shape[0]), (0, 0)))

    degp = _deg_kernel()(dst_r)
    dp0 = degp[0].reshape(N_PAD, 1)
    dp1 = degp[1].reshape(N_PAD, 1)
    hl, hr, dinv = _tc_a(dp0, dp1, x_pad, W1)
    p = _agg2_kernel()(src_r, dst_r, hl, hr)
    h2s = _tc_b(dinv, hl, hr, p[0, 0], p[0, 1], p[1, 0], p[1, 1],
                b1.reshape(1, D_HID), W2)
    q = _agg1_kernel()(src_r, dst_r, h2s)
    y = _tc_c(dinv, h2s, q[0], q[1], b2.reshape(1, D_OUT))
    return y[:N_NODES]


def kernel(x, edge_index, W1, b1, W2, b2):
    return _run(x, edge_index, W1, b1, W2, b2)


# speedup vs baseline: 11.1936x; 1.1578x over previous
import functools

import jax
import jax.numpy as jnp
from jax import lax
from jax.experimental import pallas as pl
from jax.experimental.pallas import tpu as pltpu
from jax.experimental.pallas import tpu_sc as plsc

N_NODES = 10000
D_IN = 128
D_HID = 128
D_OUT = 64
DW = 64

NC = 2
NS = 16
LANES = 16
NW = NC * NS
CHUNK = 128
CHUNKS_PER_TILE = 80
E_TILE = CHUNK * CHUNKS_PER_TILE
E_PAD = NW * E_TILE
N_PAD = 10240
SLAB = N_PAD // NS


@functools.lru_cache(maxsize=None)
def _get_mesh():
    return plsc.VectorSubcoreMesh(core_axis_name="c", subcore_axis_name="s",
                                  num_cores=NC, num_subcores=NS)


def _deg_body(dst_hbm, out_hbm, dst_v, zbuf, obuf, deg_sh):
    c = lax.axis_index("c")
    s = lax.axis_index("s")
    wid = s * NC + c
    zeros = jnp.zeros((LANES,), jnp.float32)
    ones = jnp.full((LANES,), 1.0, jnp.float32)
    for j in range(CHUNK // LANES):
        zbuf[pl.ds(j * LANES, LANES)] = zeros
        obuf[pl.ds(j * LANES, LANES)] = ones

    def zslab(i, carry):
        pltpu.sync_copy(zbuf, deg_sh.at[pl.ds(s * SLAB + i * CHUNK, CHUNK)])
        return carry

    lax.fori_loop(0, SLAB // CHUNK, zslab, 0)
    pltpu.sync_copy(dst_hbm.at[wid], dst_v)
    plsc.subcore_barrier()

    def body(k, carry):
        pltpu.sync_copy(obuf, deg_sh.at[dst_v.at[k]], add=True)
        return carry

    lax.fori_loop(0, CHUNKS_PER_TILE, body, 0)
    plsc.subcore_barrier()
    pltpu.sync_copy(deg_sh.at[pl.ds(s * SLAB, SLAB)],
                    out_hbm.at[c, pl.ds(s * SLAB, SLAB)])


@functools.lru_cache(maxsize=None)
def _deg_kernel():
    return pl.kernel(
        _deg_body,
        out_type=jax.ShapeDtypeStruct((NC, N_PAD), jnp.float32),
        mesh=_get_mesh(),
        scratch_types=[
            pltpu.VMEM((CHUNKS_PER_TILE, CHUNK), jnp.int32),
            pltpu.VMEM((CHUNK,), jnp.float32),
            pltpu.VMEM((CHUNK,), jnp.float32),
            pltpu.VMEM_SHARED((N_PAD,), jnp.float32),
        ],
        compiler_params=pltpu.CompilerParams(use_tc_tiling_on_sc=False),
    )


G = 2
NG2 = CHUNKS_PER_TILE // (2 * G)


def _fill_zbuf(zbuf):
    zeros = jnp.zeros((LANES,), jnp.float32)

    def zrow(i, carry):
        for j in range(DW // LANES):
            zbuf[i, pl.ds(j * LANES, LANES)] = zeros
        return carry

    lax.fori_loop(0, CHUNK, zrow, 0)


def _agg_pass(c, s, src_v, dst_v, gbuf, zbuf, acc_sh, sems, h_hbm, out_hbm):
    sem_g0, sem_g1, sem_s0, sem_s1 = sems

    def zslab(i, carry):
        pltpu.sync_copy(zbuf,
                        acc_sh.at[pl.ds(s * SLAB + i * CHUNK, CHUNK)])
        return carry

    lax.fori_loop(0, SLAB // CHUNK, zslab, 0)
    plsc.subcore_barrier()

    def gather(k, b, j, sem):
        return pltpu.async_copy(h_hbm.at[src_v.at[k]], gbuf.at[b, j], sem)

    def scatter(k, b, j, sem):
        return pltpu.async_copy(gbuf.at[b, j], acc_sh.at[dst_v.at[k]], sem,
                                add=True)

    def wait_gather(k, b, j, sem):
        pltpu.make_async_copy(h_hbm.at[src_v.at[k]], gbuf.at[b, j],
                              sem).wait()

    def wait_scatter(k, b, j, sem):
        pltpu.make_async_copy(gbuf.at[b, j], acc_sh.at[dst_v.at[k]],
                              sem).wait()

    for j in range(G):
        gather(j, 0, j, sem_g0)
    for j in range(G):
        gather(G + j, 1, j, sem_g1)

    def half(i, bank, g, sem_g, sem_s):
        base = g * G
        for j in range(G):
            wait_gather(base + j, bank, j, sem_g)
        for j in range(G):
            scatter(base + j, bank, j, sem_s)
        for j in range(G):
            wait_scatter(base + j, bank, j, sem_s)

        @pl.when(i < NG2 - 1)
        def _():
            nbase = base + 2 * G
            for j in range(G):
                gather(nbase + j, bank, j, sem_g)

    def body(i, carry):
        half(i, 0, 2 * i, sem_g0, sem_s0)
        half(i, 1, 2 * i + 1, sem_g1, sem_s1)
        return carry

    lax.fori_loop(0, NG2, body, 0)
    plsc.subcore_barrier()

    def wout(i, carry):
        pltpu.sync_copy(
            acc_sh.at[pl.ds(s * SLAB + i * CHUNK, CHUNK)],
            out_hbm.at[c, pl.ds(s * SLAB + i * CHUNK, CHUNK), :])
        return carry

    lax.fori_loop(0, SLAB // CHUNK, wout, 0)


def _agg2_body(src_hbm, dst_hbm, hl_hbm, hr_hbm, out_hbm,
               src_v, dst_v, gbuf, zbuf, acc_sh, *sems):
    c = lax.axis_index("c")
    s = lax.axis_index("s")
    wid = s * NC + c
    _fill_zbuf(zbuf)
    pltpu.sync_copy(src_hbm.at[wid], src_v)
    pltpu.sync_copy(dst_hbm.at[wid], dst_v)
    _agg_pass(c, s, src_v, dst_v, gbuf, zbuf, acc_sh, sems, hl_hbm,
              out_hbm.at[0])
    plsc.subcore_barrier()
    _agg_pass(c, s, src_v, dst_v, gbuf, zbuf, acc_sh, sems, hr_hbm,
              out_hbm.at[1])


def _agg1_body(src_hbm, dst_hbm, h_hbm, out_hbm,
               src_v, dst_v, gbuf, zbuf, acc_sh, *sems):
    c = lax.axis_index("c")
    s = lax.axis_index("s")
    wid = s * NC + c
    _fill_zbuf(zbuf)
    pltpu.sync_copy(src_hbm.at[wid], src_v)
    pltpu.sync_copy(dst_hbm.at[wid], dst_v)
    _agg_pass(c, s, src_v, dst_v, gbuf, zbuf, acc_sh, sems, h_hbm, out_hbm)


_AGG_SCRATCH = (
    pltpu.VMEM((CHUNKS_PER_TILE, CHUNK), jnp.int32),
    pltpu.VMEM((CHUNKS_PER_TILE, CHUNK), jnp.int32),
    pltpu.VMEM((2, G, CHUNK, DW), jnp.float32),
    pltpu.VMEM((CHUNK, DW), jnp.float32),
    pltpu.VMEM_SHARED((N_PAD, DW), jnp.float32),
    pltpu.SemaphoreType.DMA,
    pltpu.SemaphoreType.DMA,
    pltpu.SemaphoreType.DMA,
    pltpu.SemaphoreType.DMA,
)


@functools.lru_cache(maxsize=None)
def _agg2_kernel():
    return pl.kernel(
        _agg2_body,
        out_type=jax.ShapeDtypeStruct((2, NC, N_PAD, DW), jnp.float32),
        mesh=_get_mesh(),
        scratch_types=list(_AGG_SCRATCH),
        compiler_params=pltpu.CompilerParams(use_tc_tiling_on_sc=False),
    )


@functools.lru_cache(maxsize=None)
def _agg1_kernel():
    return pl.kernel(
        _agg1_body,
        out_type=jax.ShapeDtypeStruct((NC, N_PAD, DW), jnp.float32),
        mesh=_get_mesh(),
        scratch_types=list(_AGG_SCRATCH),
        compiler_params=pltpu.CompilerParams(use_tc_tiling_on_sc=False),
    )


BLK = 1024
GRID = N_PAD // BLK


def _tc_a_body(dp0_ref, dp1_ref, x_ref, w1_ref, hl_ref, hr_ref, dinv_ref):
    dinv = lax.rsqrt(dp0_ref[...] + dp1_ref[...] + 1.0)
    h = jnp.dot(x_ref[...], w1_ref[...],
                preferred_element_type=jnp.float32) * dinv
    hl_ref[...] = h[:, :DW]
    hr_ref[...] = h[:, DW:]
    dinv_ref[...] = dinv


def _tc_b_body(dinv_ref, hl_ref, hr_ref, pl0_ref, pl1_ref, pr0_ref, pr1_ref,
               b1_ref, w2_ref, h2s_ref):
    dinv = dinv_ref[...]
    zl = (pl0_ref[...] + pl1_ref[...] + hl_ref[...]) * dinv
    zr = (pr0_ref[...] + pr1_ref[...] + hr_ref[...]) * dinv
    z = jnp.maximum(jnp.concatenate([zl, zr], axis=1) + b1_ref[...], 0.0)
    h2s_ref[...] = jnp.dot(z, w2_ref[...],
                           preferred_element_type=jnp.float32) * dinv


def _tc_c_body(dinv_ref, h2s_ref, q0_ref, q1_ref, b2_ref, y_ref):
    t = (q0_ref[...] + q1_ref[...] + h2s_ref[...]) * dinv_ref[...] \
        + b2_ref[...]
    y_ref[...] = jax.nn.sigmoid(t)


def _row_spec(d):
    return pl.BlockSpec((BLK, d), lambda i: (i, 0))


def _full_spec(r, cdim):
    return pl.BlockSpec((r, cdim), lambda i: (0, 0))


_tc_a = pl.pallas_call(
    _tc_a_body,
    grid=(GRID,),
    in_specs=[_row_spec(1), _row_spec(1), _row_spec(D_IN),
              _full_spec(D_IN, D_HID)],
    out_specs=[_row_spec(DW), _row_spec(DW), _row_spec(1)],
    out_shape=[jax.ShapeDtypeStruct((N_PAD, DW), jnp.float32),
               jax.ShapeDtypeStruct((N_PAD, DW), jnp.float32),
               jax.ShapeDtypeStruct((N_PAD, 1), jnp.float32)],
)

_tc_b = pl.pallas_call(
    _tc_b_body,
    grid=(GRID,),
    in_specs=[_row_spec(1)] + [_row_spec(DW)] * 6 +
             [_full_spec(1, D_HID), _full_spec(D_HID, D_OUT)],
    out_specs=_row_spec(D_OUT),
    out_shape=jax.ShapeDtypeStruct((N_PAD, D_OUT), jnp.float32),
)

_tc_c = pl.pallas_call(
    _tc_c_body,
    grid=(GRID,),
    in_specs=[_row_spec(1), _row_spec(D_OUT), _row_spec(D_OUT),
              _row_spec(D_OUT), _full_spec(1, D_OUT)],
    out_specs=_row_spec(D_OUT),
    out_shape=jax.ShapeDtypeStruct((N_PAD, D_OUT), jnp.float32),
)


@jax.jit
def _run(x, edge_index, W1, b1, W2, b2):
    n_edges = edge_index.shape[1]
    src = edge_index[0].astype(jnp.int32)
    dst = edge_index[1].astype(jnp.int32)
    pad = E_PAD - n_edges
    src = jnp.concatenate([src, jnp.zeros((pad,), jnp.int32)])
    dst = jnp.concatenate([dst, jnp.full((pad,), N_NODES, jnp.int32)])
    src_r = src.reshape(NW, CHUNKS_PER_TILE, CHUNK)
    dst_r = dst.reshape(NW, CHUNKS_PER_TILE, CHUNK)
    x_pad = jnp.pad(x, ((0, N_PAD - x.shape[0]), (0, 0)))

    degp = _deg_kernel()(dst_r)
    dp0 = degp[0].reshape(N_PAD, 1)
    dp1 = degp[1].reshape(N_PAD, 1)
    hl, hr, dinv = _tc_a(dp0, dp1, x_pad, W1)
    p = _agg2_kernel()(src_r, dst_r, hl, hr)
    h2s = _tc_b(dinv, hl, hr, p[0, 0], p[0, 1], p[1, 0], p[1, 1],
                b1.reshape(1, D_HID), W2)
    q = _agg1_kernel()(src_r, dst_r, h2s)
    y = _tc_c(dinv, h2s, q[0], q[1], b2.reshape(1, D_OUT))
    return y[:N_NODES]


def kernel(x, edge_index, W1, b1, W2, b2):
    return _run(x, edge_index, W1, b1, W2, b2)


# speedup vs baseline: 23.7278x; 2.1198x over previous
import functools

import jax
import jax.numpy as jnp
from jax import lax
from jax.experimental import pallas as pl
from jax.experimental.pallas import tpu as pltpu
from jax.experimental.pallas import tpu_sc as plsc

N_NODES = 10000
D_IN = 128
D_HID = 128
D_OUT = 64
DW = 64

NC = 2
NS = 16
LANES = 16
NW = NC * NS
CHUNK = 128
CHUNKS_PER_TILE = 80
E_TILE = CHUNK * CHUNKS_PER_TILE
E_PAD = NW * E_TILE
N_PAD = 10240
SLAB = N_PAD // NS


@functools.lru_cache(maxsize=None)
def _get_mesh():
    return plsc.VectorSubcoreMesh(core_axis_name="c", subcore_axis_name="s",
                                  num_cores=NC, num_subcores=NS)


def _deg_body(dst_hbm, out_hbm, dst_v, zbuf, obuf, deg_sh):
    c = lax.axis_index("c")
    s = lax.axis_index("s")
    wid = s * NC + c
    zeros = jnp.zeros((LANES,), jnp.float32)
    ones = jnp.full((LANES,), 1.0, jnp.float32)
    for j in range(CHUNK // LANES):
        zbuf[pl.ds(j * LANES, LANES)] = zeros
        obuf[pl.ds(j * LANES, LANES)] = ones

    def zslab(i, carry):
        pltpu.sync_copy(zbuf, deg_sh.at[pl.ds(s * SLAB + i * CHUNK, CHUNK)])
        return carry

    lax.fori_loop(0, SLAB // CHUNK, zslab, 0)
    pltpu.sync_copy(dst_hbm.at[wid], dst_v)
    plsc.subcore_barrier()

    def body(k, carry):
        pltpu.sync_copy(obuf, deg_sh.at[dst_v.at[k]], add=True)
        return carry

    lax.fori_loop(0, CHUNKS_PER_TILE, body, 0)
    plsc.subcore_barrier()
    pltpu.sync_copy(deg_sh.at[pl.ds(s * SLAB, SLAB)],
                    out_hbm.at[c, pl.ds(s * SLAB, SLAB)])


@functools.lru_cache(maxsize=None)
def _deg_kernel():
    return pl.kernel(
        _deg_body,
        out_type=jax.ShapeDtypeStruct((NC, N_PAD), jnp.float32),
        mesh=_get_mesh(),
        scratch_types=[
            pltpu.VMEM((CHUNKS_PER_TILE, CHUNK), jnp.int32),
            pltpu.VMEM((CHUNK,), jnp.float32),
            pltpu.VMEM((CHUNK,), jnp.float32),
            pltpu.VMEM_SHARED((N_PAD,), jnp.float32),
        ],
        compiler_params=pltpu.CompilerParams(use_tc_tiling_on_sc=False),
    )


NG2 = CHUNKS_PER_TILE // 2
ZR = 32


def _fill_zbuf(zbuf):
    zeros = jnp.zeros((LANES,), jnp.float32)

    def zrow(i, carry):
        for j in range(DW // LANES):
            zbuf[i, pl.ds(j * LANES, LANES)] = zeros
        return carry

    lax.fori_loop(0, ZR, zrow, 0)


def _agg_pass(c, s, src_v, dst_v, gbuf, zbuf, h_sh, acc_sh, sems,
              h_hbm, out_hbm):
    sem_g0, sem_g1, sem_s0, sem_s1 = sems

    pltpu.sync_copy(h_hbm.at[pl.ds(s * SLAB, SLAB)],
                    h_sh.at[pl.ds(s * SLAB, SLAB)])

    def zslab(i, carry):
        pltpu.sync_copy(zbuf, acc_sh.at[pl.ds(s * SLAB + i * ZR, ZR)])
        return carry

    lax.fori_loop(0, SLAB // ZR, zslab, 0)
    plsc.subcore_barrier()

    def gather(k, b, sem):
        return pltpu.async_copy(h_sh.at[src_v.at[k]], gbuf.at[b], sem)

    def scatter(k, b, sem):
        return pltpu.async_copy(gbuf.at[b], acc_sh.at[dst_v.at[k]], sem,
                                add=True)

    def wait_gather(k, b, sem):
        pltpu.make_async_copy(h_sh.at[src_v.at[k]], gbuf.at[b], sem).wait()

    def wait_scatter(k, b, sem):
        pltpu.make_async_copy(gbuf.at[b], acc_sh.at[dst_v.at[k]],
                              sem).wait()

    gather(0, 0, sem_g0)
    gather(1, 1, sem_g1)

    def body(i, carry):
        k0 = 2 * i
        k1 = 2 * i + 1
        wait_gather(k0, 0, sem_g0)
        scatter(k0, 0, sem_s0)
        wait_gather(k1, 1, sem_g1)
        scatter(k1, 1, sem_s1)
        wait_scatter(k0, 0, sem_s0)

        @pl.when(i < NG2 - 1)
        def _():
            gather(k0 + 2, 0, sem_g0)

        wait_scatter(k1, 1, sem_s1)

        @pl.when(i < NG2 - 1)
        def _():
            gather(k1 + 2, 1, sem_g1)

        return carry

    lax.fori_loop(0, NG2, body, 0)
    plsc.subcore_barrier()

    def wout(i, carry):
        pltpu.sync_copy(
            acc_sh.at[pl.ds(s * SLAB + i * CHUNK, CHUNK)],
            out_hbm.at[c, pl.ds(s * SLAB + i * CHUNK, CHUNK), :])
        return carry

    lax.fori_loop(0, SLAB // CHUNK, wout, 0)


def _agg2_body(src_hbm, dst_hbm, hl_hbm, hr_hbm, out_hbm,
               src_v, dst_v, gbuf, zbuf, h_sh, acc_sh, *sems):
    c = lax.axis_index("c")
    s = lax.axis_index("s")
    wid = s * NC + c
    _fill_zbuf(zbuf)
    pltpu.sync_copy(src_hbm.at[wid], src_v)
    pltpu.sync_copy(dst_hbm.at[wid], dst_v)
    _agg_pass(c, s, src_v, dst_v, gbuf, zbuf, h_sh, acc_sh, sems, hl_hbm,
              out_hbm.at[0])
    plsc.subcore_barrier()
    _agg_pass(c, s, src_v, dst_v, gbuf, zbuf, h_sh, acc_sh, sems, hr_hbm,
              out_hbm.at[1])


def _agg1_body(src_hbm, dst_hbm, h_hbm, out_hbm,
               src_v, dst_v, gbuf, zbuf, h_sh, acc_sh, *sems):
    c = lax.axis_index("c")
    s = lax.axis_index("s")
    wid = s * NC + c
    _fill_zbuf(zbuf)
    pltpu.sync_copy(src_hbm.at[wid], src_v)
    pltpu.sync_copy(dst_hbm.at[wid], dst_v)
    _agg_pass(c, s, src_v, dst_v, gbuf, zbuf, h_sh, acc_sh, sems, h_hbm,
              out_hbm)


_AGG_SCRATCH = (
    pltpu.VMEM((CHUNKS_PER_TILE, CHUNK), jnp.int32),
    pltpu.VMEM((CHUNKS_PER_TILE, CHUNK), jnp.int32),
    pltpu.VMEM((2, CHUNK, DW), jnp.float32),
    pltpu.VMEM((ZR, DW), jnp.float32),
    pltpu.VMEM_SHARED((N_PAD, DW), jnp.float32),
    pltpu.VMEM_SHARED((N_PAD, DW), jnp.float32),
    pltpu.SemaphoreType.DMA,
    pltpu.SemaphoreType.DMA,
    pltpu.SemaphoreType.DMA,
    pltpu.SemaphoreType.DMA,
)


@functools.lru_cache(maxsize=None)
def _agg2_kernel():
    return pl.kernel(
        _agg2_body,
        out_type=jax.ShapeDtypeStruct((2, NC, N_PAD, DW), jnp.float32),
        mesh=_get_mesh(),
        scratch_types=list(_AGG_SCRATCH),
        compiler_params=pltpu.CompilerParams(use_tc_tiling_on_sc=False),
    )


@functools.lru_cache(maxsize=None)
def _agg1_kernel():
    return pl.kernel(
        _agg1_body,
        out_type=jax.ShapeDtypeStruct((NC, N_PAD, DW), jnp.float32),
        mesh=_get_mesh(),
        scratch_types=list(_AGG_SCRATCH),
        compiler_params=pltpu.CompilerParams(use_tc_tiling_on_sc=False),
    )


BLK = 1024
GRID = N_PAD // BLK


def _tc_a_body(dp0_ref, dp1_ref, x_ref, w1_ref, hl_ref, hr_ref, dinv_ref):
    dinv = lax.rsqrt(dp0_ref[...] + dp1_ref[...] + 1.0)
    h = jnp.dot(x_ref[...], w1_ref[...],
                preferred_element_type=jnp.float32) * dinv
    hl_ref[...] = h[:, :DW]
    hr_ref[...] = h[:, DW:]
    dinv_ref[...] = dinv


def _tc_b_body(dinv_ref, hl_ref, hr_ref, pl0_ref, pl1_ref, pr0_ref, pr1_ref,
               b1_ref, w2_ref, h2s_ref):
    dinv = dinv_ref[...]
    zl = (pl0_ref[...] + pl1_ref[...] + hl_ref[...]) * dinv
    zr = (pr0_ref[...] + pr1_ref[...] + hr_ref[...]) * dinv
    z = jnp.maximum(jnp.concatenate([zl, zr], axis=1) + b1_ref[...], 0.0)
    h2s_ref[...] = jnp.dot(z, w2_ref[...],
                           preferred_element_type=jnp.float32) * dinv


def _tc_c_body(dinv_ref, h2s_ref, q0_ref, q1_ref, b2_ref, y_ref):
    t = (q0_ref[...] + q1_ref[...] + h2s_ref[...]) * dinv_ref[...] \
        + b2_ref[...]
    y_ref[...] = jax.nn.sigmoid(t)


def _row_spec(d):
    return pl.BlockSpec((BLK, d), lambda i: (i, 0))


def _full_spec(r, cdim):
    return pl.BlockSpec((r, cdim), lambda i: (0, 0))


_tc_a = pl.pallas_call(
    _tc_a_body,
    grid=(GRID,),
    in_specs=[_row_spec(1), _row_spec(1), _row_spec(D_IN),
              _full_spec(D_IN, D_HID)],
    out_specs=[_row_spec(DW), _row_spec(DW), _row_spec(1)],
    out_shape=[jax.ShapeDtypeStruct((N_PAD, DW), jnp.float32),
               jax.ShapeDtypeStruct((N_PAD, DW), jnp.float32),
               jax.ShapeDtypeStruct((N_PAD, 1), jnp.float32)],
)

_tc_b = pl.pallas_call(
    _tc_b_body,
    grid=(GRID,),
    in_specs=[_row_spec(1)] + [_row_spec(DW)] * 6 +
             [_full_spec(1, D_HID), _full_spec(D_HID, D_OUT)],
    out_specs=_row_spec(D_OUT),
    out_shape=jax.ShapeDtypeStruct((N_PAD, D_OUT), jnp.float32),
)

_tc_c = pl.pallas_call(
    _tc_c_body,
    grid=(GRID,),
    in_specs=[_row_spec(1), _row_spec(D_OUT), _row_spec(D_OUT),
              _row_spec(D_OUT), _full_spec(1, D_OUT)],
    out_specs=_row_spec(D_OUT),
    out_shape=jax.ShapeDtypeStruct((N_PAD, D_OUT), jnp.float32),
)


@jax.jit
def _run(x, edge_index, W1, b1, W2, b2):
    n_edges = edge_index.shape[1]
    src = edge_index[0].astype(jnp.int32)
    dst = edge_index[1].astype(jnp.int32)
    pad = E_PAD - n_edges
    src = jnp.concatenate([src, jnp.zeros((pad,), jnp.int32)])
    dst = jnp.concatenate([dst, jnp.full((pad,), N_NODES, jnp.int32)])
    src_r = src.reshape(NW, CHUNKS_PER_TILE, CHUNK)
    dst_r = dst.reshape(NW, CHUNKS_PER_TILE, CHUNK)
    x_pad = jnp.pad(x, ((0, N_PAD - x.shape[0]), (0, 0)))

    degp = _deg_kernel()(dst_r)
    dp0 = degp[0].reshape(N_PAD, 1)
    dp1 = degp[1].reshape(N_PAD, 1)
    hl, hr, dinv = _tc_a(dp0, dp1, x_pad, W1)
    p = _agg2_kernel()(src_r, dst_r, hl, hr)
    h2s = _tc_b(dinv, hl, hr, p[0, 0], p[0, 1], p[1, 0], p[1, 1],
                b1.reshape(1, D_HID), W2)
    q = _agg1_kernel()(src_r, dst_r, h2s)
    y = _tc_c(dinv, h2s, q[0], q[1], b2.reshape(1, D_OUT))
    return y[:N_NODES]


def kernel(x, edge_index, W1, b1, W2, b2):
    return _run(x, edge_index, W1, b1, W2, b2)


# speedup vs baseline: 24.0682x; 1.0143x over previous
import functools

import jax
import jax.numpy as jnp
from jax import lax
from jax.experimental import pallas as pl
from jax.experimental.pallas import tpu as pltpu
from jax.experimental.pallas import tpu_sc as plsc

N_NODES = 10000
D_IN = 128
D_HID = 128
D_OUT = 64
DW = 64

NC = 2
NS = 16
LANES = 16
NW = NC * NS
CHUNK = 128
CHUNKS_PER_TILE = 80
E_TILE = CHUNK * CHUNKS_PER_TILE
E_PAD = NW * E_TILE
N_PAD = 10240
SLAB = N_PAD // NS


@functools.lru_cache(maxsize=None)
def _get_mesh():
    return plsc.VectorSubcoreMesh(core_axis_name="c", subcore_axis_name="s",
                                  num_cores=NC, num_subcores=NS)


def _deg_body(dst_hbm, out_hbm, dst_v, zbuf, obuf, deg_sh):
    c = lax.axis_index("c")
    s = lax.axis_index("s")
    wid = s * NC + c
    zeros = jnp.zeros((LANES,), jnp.float32)
    ones = jnp.full((LANES,), 1.0, jnp.float32)
    for j in range(CHUNK // LANES):
        zbuf[pl.ds(j * LANES, LANES)] = zeros
        obuf[pl.ds(j * LANES, LANES)] = ones

    def zslab(i, carry):
        pltpu.sync_copy(zbuf, deg_sh.at[pl.ds(s * SLAB + i * CHUNK, CHUNK)])
        return carry

    lax.fori_loop(0, SLAB // CHUNK, zslab, 0)
    pltpu.sync_copy(dst_hbm.at[wid], dst_v)
    plsc.subcore_barrier()

    def body(k, carry):
        pltpu.sync_copy(obuf, deg_sh.at[dst_v.at[k]], add=True)
        return carry

    lax.fori_loop(0, CHUNKS_PER_TILE, body, 0)
    plsc.subcore_barrier()
    pltpu.sync_copy(deg_sh.at[pl.ds(s * SLAB, SLAB)],
                    out_hbm.at[c, pl.ds(s * SLAB, SLAB)])


@functools.lru_cache(maxsize=None)
def _deg_kernel():
    return pl.kernel(
        _deg_body,
        out_type=jax.ShapeDtypeStruct((NC, N_PAD), jnp.float32),
        mesh=_get_mesh(),
        scratch_types=[
            pltpu.VMEM((CHUNKS_PER_TILE, CHUNK), jnp.int32),
            pltpu.VMEM((CHUNK,), jnp.float32),
            pltpu.VMEM((CHUNK,), jnp.float32),
            pltpu.VMEM_SHARED((N_PAD,), jnp.float32),
        ],
        compiler_params=pltpu.CompilerParams(use_tc_tiling_on_sc=False),
    )


NG2 = CHUNKS_PER_TILE // 2
ZR = 32


def _fill_zbuf(zbuf):
    zeros = jnp.zeros((LANES,), jnp.float32)

    def zrow(i, carry):
        for j in range(DW // LANES):
            zbuf[i, pl.ds(j * LANES, LANES)] = zeros
        return carry

    lax.fori_loop(0, ZR, zrow, 0)


def _agg_pass(c, s, src_v, dst_v, gbuf, zbuf, h_sh, acc_sh, sems,
              h_hbm, out_hbm):
    sem_g0, sem_g1, sem_s0, sem_s1 = sems

    pltpu.sync_copy(h_hbm.at[pl.ds(s * SLAB, SLAB)],
                    h_sh.at[pl.ds(s * SLAB, SLAB)])

    def zslab(i, carry):
        pltpu.sync_copy(zbuf, acc_sh.at[pl.ds(s * SLAB + i * ZR, ZR)])
        return carry

    lax.fori_loop(0, SLAB // ZR, zslab, 0)
    plsc.subcore_barrier()

    def gather(k, b, sem):
        return pltpu.async_copy(h_sh.at[src_v.at[k]], gbuf.at[b], sem)

    def scatter(k, b, sem):
        return pltpu.async_copy(gbuf.at[b], acc_sh.at[dst_v.at[k]], sem,
                                add=True)

    def wait_gather(k, b, sem):
        pltpu.make_async_copy(h_sh.at[src_v.at[k]], gbuf.at[b], sem).wait()

    def wait_scatter(k, b, sem):
        pltpu.make_async_copy(gbuf.at[b], acc_sh.at[dst_v.at[k]],
                              sem).wait()

    gather(0, 0, sem_g0)
    gather(1, 1, sem_g1)

    def body(i, carry):
        k0 = 2 * i
        k1 = 2 * i + 1
        wait_gather(k0, 0, sem_g0)
        scatter(k0, 0, sem_s0)
        wait_gather(k1, 1, sem_g1)
        scatter(k1, 1, sem_s1)
        wait_scatter(k0, 0, sem_s0)

        @pl.when(i < NG2 - 1)
        def _():
            gather(k0 + 2, 0, sem_g0)

        wait_scatter(k1, 1, sem_s1)

        @pl.when(i < NG2 - 1)
        def _():
            gather(k1 + 2, 1, sem_g1)

        return carry

    lax.fori_loop(0, NG2, body, 0)
    plsc.subcore_barrier()

    def wout(i, carry):
        pltpu.sync_copy(
            acc_sh.at[pl.ds(s * SLAB + i * CHUNK, CHUNK)],
            out_hbm.at[c, pl.ds(s * SLAB + i * CHUNK, CHUNK), :])
        return carry

    lax.fori_loop(0, SLAB // CHUNK, wout, 0)


def _agg1_body(src_hbm, dst_hbm, h_hbm, out_hbm,
               src_v, dst_v, gbuf, zbuf, h_sh, acc_sh, *sems):
    c = lax.axis_index("c")
    s = lax.axis_index("s")
    wid = s * NC + c
    _fill_zbuf(zbuf)
    pltpu.sync_copy(src_hbm.at[wid], src_v)
    pltpu.sync_copy(dst_hbm.at[wid], dst_v)
    _agg_pass(c, s, src_v, dst_v, gbuf, zbuf, h_sh, acc_sh, sems, h_hbm,
              out_hbm)


_AGG_SCRATCH = (
    pltpu.VMEM((CHUNKS_PER_TILE, CHUNK), jnp.int32),
    pltpu.VMEM((CHUNKS_PER_TILE, CHUNK), jnp.int32),
    pltpu.VMEM((2, CHUNK, DW), jnp.float32),
    pltpu.VMEM((ZR, DW), jnp.float32),
    pltpu.VMEM_SHARED((N_PAD, DW), jnp.float32),
    pltpu.VMEM_SHARED((N_PAD, DW), jnp.float32),
    pltpu.SemaphoreType.DMA,
    pltpu.SemaphoreType.DMA,
    pltpu.SemaphoreType.DMA,
    pltpu.SemaphoreType.DMA,
)


@functools.lru_cache(maxsize=None)
def _agg1_kernel():
    return pl.kernel(
        _agg1_body,
        out_type=jax.ShapeDtypeStruct((NC, N_PAD, DW), jnp.float32),
        mesh=_get_mesh(),
        scratch_types=list(_AGG_SCRATCH),
        compiler_params=pltpu.CompilerParams(use_tc_tiling_on_sc=False),
    )


BLK = 1024
GRID = N_PAD // BLK
BLK_C = 400
GRID_C = N_NODES // BLK_C


def _tc_a1_body(x_ref, w1_ref, h1_ref):
    h1_ref[...] = jnp.dot(x_ref[...], w1_ref[...],
                          preferred_element_type=jnp.float32)


def _tc_a2_body(dp0_ref, dp1_ref, h1_ref, hl_ref, hr_ref, dinv_ref):
    dinv = lax.rsqrt(dp0_ref[...] + dp1_ref[...] + 1.0)
    h = h1_ref[...] * dinv
    hl_ref[...] = h[:, :DW]
    hr_ref[...] = h[:, DW:]
    dinv_ref[...] = dinv


def _tc_b1_body(dinv_ref, hl_ref, pa_ref, pb_ref, b1l_ref, w2a_ref,
                h2sa_ref):
    zl = jnp.maximum(
        (pa_ref[0] + pb_ref[0] + hl_ref[...]) * dinv_ref[...]
        + b1l_ref[...], 0.0)
    h2sa_ref[...] = jnp.dot(zl, w2a_ref[...],
                            preferred_element_type=jnp.float32)


def _tc_b2_body(dinv_ref, hr_ref, pa_ref, pb_ref, b1r_ref, w2b_ref,
                h2sa_ref, h2s_ref):
    dinv = dinv_ref[...]
    zr = jnp.maximum(
        (pa_ref[0] + pb_ref[0] + hr_ref[...]) * dinv + b1r_ref[...], 0.0)
    h2s_ref[...] = (h2sa_ref[...] +
                    jnp.dot(zr, w2b_ref[...],
                            preferred_element_type=jnp.float32)) * dinv


def _tc_c_body(dinv_ref, h2s_ref, qa_ref, qb_ref, b2_ref, y_ref):
    t = (qa_ref[0] + qb_ref[0] + h2s_ref[...]) * dinv_ref[...] \
        + b2_ref[...]
    y_ref[...] = jax.nn.sigmoid(t)


def _row_spec(d, blk=BLK):
    return pl.BlockSpec((blk, d), lambda i: (i, 0))


def _full_spec(r, cdim):
    return pl.BlockSpec((r, cdim), lambda i: (0, 0))


def _part_spec(core, blk=BLK):
    return pl.BlockSpec((1, blk, DW), lambda i, c=core: (c, i, 0))


_tc_a1 = pl.pallas_call(
    _tc_a1_body,
    grid=(GRID,),
    in_specs=[_row_spec(D_IN), _full_spec(D_IN, D_HID)],
    out_specs=_row_spec(D_HID),
    out_shape=jax.ShapeDtypeStruct((N_PAD, D_HID), jnp.float32),
)

_tc_a2 = pl.pallas_call(
    _tc_a2_body,
    grid=(GRID,),
    in_specs=[_row_spec(1), _row_spec(1), _row_spec(D_HID)],
    out_specs=[_row_spec(DW), _row_spec(DW), _row_spec(1)],
    out_shape=[jax.ShapeDtypeStruct((N_PAD, DW), jnp.float32),
               jax.ShapeDtypeStruct((N_PAD, DW), jnp.float32),
               jax.ShapeDtypeStruct((N_PAD, 1), jnp.float32)],
)

_tc_b1 = pl.pallas_call(
    _tc_b1_body,
    grid=(GRID,),
    in_specs=[_row_spec(1), _row_spec(DW), _part_spec(0), _part_spec(1),
              _full_spec(1, DW), _full_spec(DW, D_OUT)],
    out_specs=_row_spec(D_OUT),
    out_shape=jax.ShapeDtypeStruct((N_PAD, D_OUT), jnp.float32),
)

_tc_b2 = pl.pallas_call(
    _tc_b2_body,
    grid=(GRID,),
    in_specs=[_row_spec(1), _row_spec(DW), _part_spec(0), _part_spec(1),
              _full_spec(1, DW), _full_spec(DW, D_OUT), _row_spec(D_OUT)],
    out_specs=_row_spec(D_OUT),
    out_shape=jax.ShapeDtypeStruct((N_PAD, D_OUT), jnp.float32),
)

_tc_c = pl.pallas_call(
    _tc_c_body,
    grid=(GRID_C,),
    in_specs=[_row_spec(1, BLK_C), _row_spec(D_OUT, BLK_C),
              _part_spec(0, BLK_C), _part_spec(1, BLK_C),
              _full_spec(1, D_OUT)],
    out_specs=_row_spec(D_OUT, BLK_C),
    out_shape=jax.ShapeDtypeStruct((N_NODES, D_OUT), jnp.float32),
)


@jax.jit
def _run(x, edge_index, W1, b1, W2, b2):
    n_edges = edge_index.shape[1]
    src = edge_index[0].astype(jnp.int32)
    dst = edge_index[1].astype(jnp.int32)
    pad = E_PAD - n_edges
    src = jnp.concatenate([src, jnp.zeros((pad,), jnp.int32)])
    dst = jnp.concatenate([dst, jnp.full((pad,), N_NODES, jnp.int32)])
    src_r = src.reshape(NW, CHUNKS_PER_TILE, CHUNK)
    dst_r = dst.reshape(NW, CHUNKS_PER_TILE, CHUNK)
    x_pad = jnp.pad(x, ((0, N_PAD - x.shape[0]), (0, 0)))

    degp = _deg_kernel()(dst_r)
    h1 = _tc_a1(x_pad, W1)
    dp0 = degp[0].reshape(N_PAD, 1)
    dp1 = degp[1].reshape(N_PAD, 1)
    hl, hr, dinv = _tc_a2(dp0, dp1, h1)
    agg = _agg1_kernel()
    p_l = agg(src_r, dst_r, hl)
    p_r = agg(src_r, dst_r, hr)
    h2sa = _tc_b1(dinv, hl, p_l, p_l, b1[:DW].reshape(1, DW), W2[:DW])
    h2s = _tc_b2(dinv, hr, p_r, p_r, b1[DW:].reshape(1, DW), W2[DW:], h2sa)
    q = agg(src_r, dst_r, h2s)
    return _tc_c(dinv, h2s, q, q, b2.reshape(1, D_OUT))


def kernel(x, edge_index, W1, b1, W2, b2):
    return _run(x, edge_index, W1, b1, W2, b2)
